# baseline (speedup 1.0000x reference)
import jax
import jax.numpy as jnp
from jax import lax
from jax.experimental import pallas as pl
from jax.experimental.pallas import tpu as pltpu
from jax.experimental.pallas import tpu_sc as plsc

_NUM_ROWS = 16384 * 50
_DIM = 32
_NC, _NS = 2, 16
_NW = _NC * _NS
_PER_W = _NUM_ROWS // _NW
_CHUNK = 800
_NCHUNK = _PER_W // _CHUNK
_K = 4


def _body(table_hbm, idx_hbm, out_hbm, idx_v,
          rows_0, rows_1, rows_2, rows_3,
          g_sem_0, g_sem_1, g_sem_2, g_sem_3, s_sem):
    wid = lax.axis_index("s") * _NC + lax.axis_index("c")
    base = wid * _PER_W
    rows = (rows_0, rows_1, rows_2, rows_3)
    sems = (g_sem_0, g_sem_1, g_sem_2, g_sem_3)

    pltpu.sync_copy(idx_hbm.at[wid], idx_v)

    def group(g, carry):
        for k in range(_K):
            pltpu.make_async_copy(table_hbm.at[idx_v.at[g * _K + k]],
                                  rows[k], sems[k]).start()
        for k in range(_K):
            pltpu.make_async_copy(table_hbm.at[idx_v.at[g * _K + k]],
                                  rows[k], sems[k]).wait()
        return carry

    lax.fori_loop(0, _NCHUNK // _K, group, 0)
    pltpu.async_copy(rows_0, out_hbm.at[pl.ds(base, _CHUNK)], s_sem).wait()


_gather_call = pl.kernel(
    _body,
    out_type=jax.ShapeDtypeStruct((_NUM_ROWS, _DIM), jnp.float32),
    mesh=plsc.VectorSubcoreMesh(core_axis_name="c", subcore_axis_name="s"),
    scratch_types=[
        pltpu.VMEM((_NCHUNK, _CHUNK), jnp.int32),
        pltpu.VMEM((_CHUNK, _DIM), jnp.float32),
        pltpu.VMEM((_CHUNK, _DIM), jnp.float32),
        pltpu.VMEM((_CHUNK, _DIM), jnp.float32),
        pltpu.VMEM((_CHUNK, _DIM), jnp.float32),
        pltpu.SemaphoreType.DMA,
        pltpu.SemaphoreType.DMA,
        pltpu.SemaphoreType.DMA,
        pltpu.SemaphoreType.DMA,
        pltpu.SemaphoreType.DMA,
    ],
    compiler_params=pltpu.CompilerParams(use_tc_tiling_on_sc=False),
)


def kernel(indices, table):
    flat_idx = indices.reshape(_NW, _NCHUNK, _CHUNK).astype(jnp.int32)
    out = _gather_call(table, flat_idx)
    return out.reshape(indices.shape + (_DIM,))
